# baseline (speedup 1.0000x reference)
import functools

import jax
import jax.numpy as jnp
from jax import lax
from jax.experimental import pallas as pl
from jax.experimental.pallas import tpu as pltpu
from jax.experimental.pallas import tpu_sc as plsc

D = 1024
E = 8
K = 2
T = 2048
CAP = 320
A = T * K
NW = 32
CHUNK = A // NW
NV = CHUNK // 16
TPW = T // NW
TRASH = E * CAP
XG_ROWS = (E + 1) * CAP


def _sc_route_body(routes_hbm, x_hbm, xg_hbm, comb_hbm, accf_hbm,
                   all_v, comb_v, acc_v,
                   it0, it1, it2, it3, is0, is1, is2, is3,
                   bufa, bufb, h_v,
                   sga, sgb, ssa, ssb):
    core = lax.axis_index("c")
    sid = lax.axis_index("s")
    cid = core * 16 + sid
    base = cid * CHUNK
    lane = lax.iota(jnp.int32, 16)
    its = [it0, it1, it2, it3]
    iss = [is0, is1, is2, is3]

    one16 = jnp.full((16,), 1, jnp.int32)
    zero16 = jnp.zeros((16,), jnp.int32)
    for v in range(NV):
        tok = lax.shift_right_logical(
            jnp.full((16,), base + v * 16, jnp.int32) + lane, one16)
        its[v // 2][pl.ds((v % 2) * 16, 16)] = tok

    pltpu.sync_copy(routes_hbm, all_v)

    for l in range(16):
        h_v[pl.ds(l * 16, 16)] = zero16
    lanebank = lane * jnp.full((16,), 16, jnp.int32)
    basev = jnp.full((16,), base, jnp.int32)
    for v in range(A // 16):
        rv = all_v[pl.ds(v * 16, 16)]
        mv = jnp.full((16,), v * 16, jnp.int32) < basev
        plsc.addupdate_scatter(h_v, [rv + lanebank], one16, mask=mv)
    pre = zero16
    for l in range(16):
        pre = pre + h_v[pl.ds(l * 16, 16)]

    ga = pltpu.async_copy(x_hbm.at[it0], bufa, sga)
    gb = pltpu.async_copy(x_hbm.at[it1], bufb, sgb)

    rvs = [all_v[pl.ds(base + v * 16, 16)] for v in range(NV)]
    poss = [zero16 for _ in range(NV)]
    for e in range(E):
        cvec = pre.at[jnp.full((16,), e, jnp.int32)].get(
            mode="promise_in_bounds")
        for v in range(NV):
            m = rvs[v] == jnp.full((16,), e, jnp.int32)
            m32 = jnp.where(m, one16, zero16)
            incl = jnp.cumsum(m32)
            poss[v] = jnp.where(m, cvec + incl - one16, poss[v])
            cvec = cvec + jnp.full((16,), jnp.sum(m32), jnp.int32)

    cap16 = jnp.full((16,), CAP, jnp.int32)
    onef16 = jnp.full((16,), 1.0, jnp.float32)
    zerof16 = jnp.zeros((16,), jnp.float32)
    for v in range(NV):
        pos, rv = poss[v], rvs[v]
        accept = pos < cap16
        comb = rv * cap16 + jnp.minimum(pos, cap16 - one16)
        slot = jnp.where(accept, comb, jnp.full((16,), TRASH, jnp.int32))
        comb_v[pl.ds(v * 16, 16)] = comb
        acc_v[pl.ds(v * 16, 16)] = jnp.where(accept, onef16, zerof16)
        iss[v // 2][pl.ds((v % 2) * 16, 16)] = slot

    pltpu.sync_copy(comb_v, comb_hbm.at[pl.ds(base, CHUNK)])
    pltpu.sync_copy(acc_v, accf_hbm.at[pl.ds(base, CHUNK)])
    ga.wait()
    sa = pltpu.async_copy(bufa, xg_hbm.at[is0], ssa)
    gb.wait()
    sb = pltpu.async_copy(bufb, xg_hbm.at[is1], ssb)
    sa.wait()
    ga = pltpu.async_copy(x_hbm.at[it2], bufa, sga)
    sb.wait()
    gb = pltpu.async_copy(x_hbm.at[it3], bufb, sgb)
    ga.wait()
    sa = pltpu.async_copy(bufa, xg_hbm.at[is2], ssa)
    gb.wait()
    sb = pltpu.async_copy(bufb, xg_hbm.at[is3], ssb)
    sa.wait()
    sb.wait()


_sc_route = functools.partial(
    pl.kernel,
    out_type=(
        jax.ShapeDtypeStruct((XG_ROWS, D), jnp.float32),
        jax.ShapeDtypeStruct((A,), jnp.int32),
        jax.ShapeDtypeStruct((A,), jnp.float32),
    ),
    mesh=plsc.VectorSubcoreMesh(core_axis_name="c", subcore_axis_name="s"),
    compiler_params=pltpu.CompilerParams(needs_layout_passes=False),
    scratch_types=[
        pltpu.VMEM((A,), jnp.int32),
        pltpu.VMEM((CHUNK,), jnp.int32),
        pltpu.VMEM((CHUNK,), jnp.float32),
        pltpu.VMEM((32,), jnp.int32),
        pltpu.VMEM((32,), jnp.int32),
        pltpu.VMEM((32,), jnp.int32),
        pltpu.VMEM((32,), jnp.int32),
        pltpu.VMEM((32,), jnp.int32),
        pltpu.VMEM((32,), jnp.int32),
        pltpu.VMEM((32,), jnp.int32),
        pltpu.VMEM((32,), jnp.int32),
        pltpu.VMEM((32, D), jnp.float32),
        pltpu.VMEM((32, D), jnp.float32),
        pltpu.VMEM((256,), jnp.int32),
        pltpu.SemaphoreType.DMA,
        pltpu.SemaphoreType.DMA,
        pltpu.SemaphoreType.DMA,
        pltpu.SemaphoreType.DMA,
    ],
)(_sc_route_body)


def _sc_combine_body(y_hbm, fb_hbm, comb_hbm, accf_hbm, out_hbm,
                     comb_v, acc_v, yb0, yb1, fu0, fu1, ob0, ob1,
                     sy0, sy1, sf0, sf1, so0, so1):
    wid = lax.axis_index("s") * 2 + lax.axis_index("c")
    abase = wid * CHUNK
    tbase = wid * TPW
    pltpu.sync_copy(comb_hbm.at[pl.ds(abase, CHUNK)], comb_v)
    pltpu.sync_copy(accf_hbm.at[pl.ds(abase, CHUNK)], acc_v)
    ybufs, fbufs, obufs = [yb0, yb1], [fu0, fu1], [ob0, ob1]
    syl, sfl, sol = [sy0, sy1], [sf0, sf1], [so0, so1]
    ycp, fcp, ocp = [None, None], [None, None], [None, None]
    NG = TPW // 8

    def fire(g):
        p = g % 2
        cvec = comb_v[pl.ds(g * 16, 16)]
        ycp[p] = pltpu.async_copy(y_hbm.at[cvec], ybufs[p], syl[p])
        fcp[p] = pltpu.async_copy(
            fb_hbm.at[pl.ds(tbase + g * 8, 8)], fbufs[p], sfl[p])

    fire(0)
    zerof16 = jnp.zeros((16,), jnp.float32)
    onef16 = jnp.full((16,), 1.0, jnp.float32)
    for g in range(NG):
        p = g % 2
        if g + 1 < NG:
            fire(g + 1)
        ycp[p].wait()
        fcp[p].wait()
        if ocp[p] is not None:
            ocp[p].wait()
        avec = acc_v[pl.ds(g * 16, 16)]
        ybuf, fbuf, obuf = ybufs[p], fbufs[p], obufs[p]

        def tok_body(t, _, avec=avec, ybuf=ybuf, fbuf=fbuf, obuf=obuf):
            a0v = avec.at[jnp.full((16,), 2 * t, jnp.int32)].get(
                mode="promise_in_bounds")
            a1v = avec.at[jnp.full((16,), 2 * t + 1, jnp.int32)].get(
                mode="promise_in_bounds")
            usedv = a0v + a1v
            invv = onef16 / jnp.maximum(usedv, onef16)
            a0v = a0v * invv
            a1v = a1v * invv
            routed = usedv > zerof16

            def c_body(cb, _):
                for u in range(4):
                    o = cb * 64 + u * 16
                    y0 = ybuf[2 * t, pl.ds(o, 16)]
                    y1 = ybuf[2 * t + 1, pl.ds(o, 16)]
                    fv = fbuf[t, pl.ds(o, 16)]
                    r = y0 * a0v + y1 * a1v
                    obuf[t, pl.ds(o, 16)] = jnp.where(routed, r, fv)
                return 0

            return lax.fori_loop(0, D // 64, c_body, 0)

        lax.fori_loop(0, 8, tok_body, 0)
        ocp[p] = pltpu.async_copy(
            obuf, out_hbm.at[pl.ds(tbase + g * 8, 8)], sol[p])
    ocp[0].wait()
    ocp[1].wait()


_sc_combine = functools.partial(
    pl.kernel,
    out_type=jax.ShapeDtypeStruct((T, D), jnp.float32),
    mesh=plsc.VectorSubcoreMesh(core_axis_name="c", subcore_axis_name="s"),
    compiler_params=pltpu.CompilerParams(needs_layout_passes=False),
    scratch_types=[
        pltpu.VMEM((CHUNK,), jnp.int32),
        pltpu.VMEM((CHUNK,), jnp.float32),
        pltpu.VMEM((16, D), jnp.float32),
        pltpu.VMEM((16, D), jnp.float32),
        pltpu.VMEM((8, D), jnp.float32),
        pltpu.VMEM((8, D), jnp.float32),
        pltpu.VMEM((8, D), jnp.float32),
        pltpu.VMEM((8, D), jnp.float32),
        pltpu.SemaphoreType.DMA,
        pltpu.SemaphoreType.DMA,
        pltpu.SemaphoreType.DMA,
        pltpu.SemaphoreType.DMA,
        pltpu.SemaphoreType.DMA,
        pltpu.SemaphoreType.DMA,
    ],
)(_sc_combine_body)


def _tc_fallback_body(x_ref, w1_ref, b1_ref, w2_ref, b2_ref, o_ref):
    h = jnp.dot(x_ref[...], w1_ref[...], preferred_element_type=jnp.float32)
    h = jnp.maximum(h + b1_ref[...], 0.0)
    o_ref[...] = (
        jnp.dot(h, w2_ref[...], preferred_element_type=jnp.float32)
        + b2_ref[...])


def _tc_fallback(x, fW1, fb1, fW2, fb2):
    blk = 256
    return pl.pallas_call(
        _tc_fallback_body,
        grid=(T // blk,),
        in_specs=[
            pl.BlockSpec((blk, D), lambda i: (i, 0)),
            pl.BlockSpec((D, D), lambda i: (0, 0)),
            pl.BlockSpec((1, D), lambda i: (0, 0)),
            pl.BlockSpec((D, D), lambda i: (0, 0)),
            pl.BlockSpec((1, D), lambda i: (0, 0)),
        ],
        out_specs=pl.BlockSpec((blk, D), lambda i: (i, 0)),
        out_shape=jax.ShapeDtypeStruct((T, D), jnp.float32),
    )(x, fW1, fb1.reshape(1, D), fW2, fb2.reshape(1, D))


def _tc_expert_body(xg_ref, w1_ref, b1_ref, w2_ref, b2_ref, y_ref):
    h = jnp.dot(xg_ref[...], w1_ref[0], preferred_element_type=jnp.float32)
    h = jnp.maximum(h + b1_ref[0], 0.0)
    y_ref[...] = (
        jnp.dot(h, w2_ref[0], preferred_element_type=jnp.float32)
        + b2_ref[0])


def _tc_experts(xg, W1, b1, W2, b2):
    return pl.pallas_call(
        _tc_expert_body,
        grid=(E,),
        in_specs=[
            pl.BlockSpec((CAP, D), lambda e: (e, 0)),
            pl.BlockSpec((1, D, D), lambda e: (e, 0, 0)),
            pl.BlockSpec((1, 1, D), lambda e: (e, 0, 0)),
            pl.BlockSpec((1, D, D), lambda e: (e, 0, 0)),
            pl.BlockSpec((1, 1, D), lambda e: (e, 0, 0)),
        ],
        out_specs=pl.BlockSpec((CAP, D), lambda e: (e, 0)),
        out_shape=jax.ShapeDtypeStruct((E * CAP, D), jnp.float32),
    )(xg, W1, b1.reshape(E, 1, D), W2, b2.reshape(E, 1, D))


def kernel(x, W1, b1, W2, b2, fW1, fb1, fW2, fb2, routes):
    routes_flat = routes.reshape(-1)
    xg, comb, accf = _sc_route(routes_flat, x)
    fb = _tc_fallback(x, fW1, fb1, fW2, fb2)
    y = _tc_experts(xg, W1, b1, W2, b2)
    return _sc_combine(y, fb, comb, accf)


# speedup vs baseline: 1.5428x; 1.5428x over previous
import functools

import jax
import jax.numpy as jnp
from jax import lax
from jax.experimental import pallas as pl
from jax.experimental.pallas import tpu as pltpu
from jax.experimental.pallas import tpu_sc as plsc

D = 1024
E = 8
K = 2
T = 2048
CAP = 320
A = T * K
NW = 32
CHUNK = A // NW
NV = CHUNK // 16
TPW = T // NW
TRASH = E * CAP
XG_ROWS = (E + 1) * CAP


def _sc_route_body(routes_hbm, x_hbm, xg_hbm, comb_hbm, accf_hbm,
                   all_v, comb_v, acc_v,
                   it0, it1, it2, it3, is0, is1, is2, is3,
                   bufa, bufb, h_v,
                   sga, sgb, ssa, ssb):
    core = lax.axis_index("c")
    sid = lax.axis_index("s")
    cid = core * 16 + sid
    base = cid * CHUNK
    lane = lax.iota(jnp.int32, 16)
    its = [it0, it1, it2, it3]
    iss = [is0, is1, is2, is3]

    one16 = jnp.full((16,), 1, jnp.int32)
    zero16 = jnp.zeros((16,), jnp.int32)
    for v in range(NV):
        tok = lax.shift_right_logical(
            jnp.full((16,), base + v * 16, jnp.int32) + lane, one16)
        its[v // 2][pl.ds((v % 2) * 16, 16)] = tok

    pltpu.sync_copy(routes_hbm, all_v)

    for l in range(16):
        h_v[pl.ds(l * 16, 16)] = zero16
    lanebank = lane * jnp.full((16,), 16, jnp.int32)
    basev = jnp.full((16,), base, jnp.int32)
    for v in range(A // 16):
        rv = all_v[pl.ds(v * 16, 16)]
        mv = jnp.full((16,), v * 16, jnp.int32) < basev
        plsc.addupdate_scatter(h_v, [rv + lanebank], one16, mask=mv)
    pre = zero16
    for l in range(16):
        pre = pre + h_v[pl.ds(l * 16, 16)]

    ga = pltpu.async_copy(x_hbm.at[it0], bufa, sga)
    gb = pltpu.async_copy(x_hbm.at[it1], bufb, sgb)

    rvs = [all_v[pl.ds(base + v * 16, 16)] for v in range(NV)]
    poss = [zero16 for _ in range(NV)]
    for e in range(E):
        cvec = pre.at[jnp.full((16,), e, jnp.int32)].get(
            mode="promise_in_bounds")
        for v in range(NV):
            m = rvs[v] == jnp.full((16,), e, jnp.int32)
            m32 = jnp.where(m, one16, zero16)
            incl = jnp.cumsum(m32)
            poss[v] = jnp.where(m, cvec + incl - one16, poss[v])
            cvec = cvec + jnp.full((16,), jnp.sum(m32), jnp.int32)

    cap16 = jnp.full((16,), CAP, jnp.int32)
    onef16 = jnp.full((16,), 1.0, jnp.float32)
    zerof16 = jnp.zeros((16,), jnp.float32)
    trashv = jnp.full((16,), TRASH, jnp.int32) + jnp.full((16,), cid, jnp.int32)
    for v in range(NV):
        pos, rv = poss[v], rvs[v]
        accept = pos < cap16
        comb = rv * cap16 + jnp.minimum(pos, cap16 - one16)
        slot = jnp.where(accept, comb, trashv)
        comb_v[pl.ds(v * 16, 16)] = comb
        acc_v[pl.ds(v * 16, 16)] = jnp.where(accept, onef16, zerof16)
        iss[v // 2][pl.ds((v % 2) * 16, 16)] = slot

    pltpu.sync_copy(comb_v, comb_hbm.at[pl.ds(base, CHUNK)])
    pltpu.sync_copy(acc_v, accf_hbm.at[pl.ds(base, CHUNK)])
    ga.wait()
    sa = pltpu.async_copy(bufa, xg_hbm.at[is0], ssa)
    gb.wait()
    sb = pltpu.async_copy(bufb, xg_hbm.at[is1], ssb)
    sa.wait()
    ga = pltpu.async_copy(x_hbm.at[it2], bufa, sga)
    sb.wait()
    gb = pltpu.async_copy(x_hbm.at[it3], bufb, sgb)
    ga.wait()
    sa = pltpu.async_copy(bufa, xg_hbm.at[is2], ssa)
    gb.wait()
    sb = pltpu.async_copy(bufb, xg_hbm.at[is3], ssb)
    sa.wait()
    sb.wait()


_sc_route = functools.partial(
    pl.kernel,
    out_type=(
        jax.ShapeDtypeStruct((XG_ROWS, D), jnp.float32),
        jax.ShapeDtypeStruct((A,), jnp.int32),
        jax.ShapeDtypeStruct((A,), jnp.float32),
    ),
    mesh=plsc.VectorSubcoreMesh(core_axis_name="c", subcore_axis_name="s"),
    compiler_params=pltpu.CompilerParams(needs_layout_passes=False),
    scratch_types=[
        pltpu.VMEM((A,), jnp.int32),
        pltpu.VMEM((CHUNK,), jnp.int32),
        pltpu.VMEM((CHUNK,), jnp.float32),
        pltpu.VMEM((32,), jnp.int32),
        pltpu.VMEM((32,), jnp.int32),
        pltpu.VMEM((32,), jnp.int32),
        pltpu.VMEM((32,), jnp.int32),
        pltpu.VMEM((32,), jnp.int32),
        pltpu.VMEM((32,), jnp.int32),
        pltpu.VMEM((32,), jnp.int32),
        pltpu.VMEM((32,), jnp.int32),
        pltpu.VMEM((32, D), jnp.float32),
        pltpu.VMEM((32, D), jnp.float32),
        pltpu.VMEM((256,), jnp.int32),
        pltpu.SemaphoreType.DMA,
        pltpu.SemaphoreType.DMA,
        pltpu.SemaphoreType.DMA,
        pltpu.SemaphoreType.DMA,
    ],
)(_sc_route_body)


def _sc_combine_body(y_hbm, fb_hbm, comb_hbm, accf_hbm, out_hbm,
                     comb_v, acc_v, yb0, yb1, fu0, fu1, ob0, ob1,
                     sy0, sy1, sf0, sf1, so0, so1):
    wid = lax.axis_index("s") * 2 + lax.axis_index("c")
    abase = wid * CHUNK
    tbase = wid * TPW
    pltpu.sync_copy(comb_hbm.at[pl.ds(abase, CHUNK)], comb_v)
    pltpu.sync_copy(accf_hbm.at[pl.ds(abase, CHUNK)], acc_v)
    ybufs, fbufs, obufs = [yb0, yb1], [fu0, fu1], [ob0, ob1]
    syl, sfl, sol = [sy0, sy1], [sf0, sf1], [so0, so1]
    ycp, fcp, ocp = [None, None], [None, None], [None, None]
    NG = TPW // 8

    def fire(g):
        p = g % 2
        cvec = comb_v[pl.ds(g * 16, 16)]
        ycp[p] = pltpu.async_copy(y_hbm.at[cvec], ybufs[p], syl[p])
        fcp[p] = pltpu.async_copy(
            fb_hbm.at[pl.ds(tbase + g * 8, 8)], fbufs[p], sfl[p])

    fire(0)
    zerof16 = jnp.zeros((16,), jnp.float32)
    onef16 = jnp.full((16,), 1.0, jnp.float32)
    for g in range(NG):
        p = g % 2
        if g + 1 < NG:
            fire(g + 1)
        ycp[p].wait()
        fcp[p].wait()
        if ocp[p] is not None:
            ocp[p].wait()
        avec = acc_v[pl.ds(g * 16, 16)]
        ybuf, fbuf, obuf = ybufs[p], fbufs[p], obufs[p]

        def tok_body(t, _, avec=avec, ybuf=ybuf, fbuf=fbuf, obuf=obuf):
            a0v = avec.at[jnp.full((16,), 2 * t, jnp.int32)].get(
                mode="promise_in_bounds")
            a1v = avec.at[jnp.full((16,), 2 * t + 1, jnp.int32)].get(
                mode="promise_in_bounds")
            usedv = a0v + a1v
            invv = onef16 / jnp.maximum(usedv, onef16)
            a0v = a0v * invv
            a1v = a1v * invv
            routed = usedv > zerof16

            def c_body(cb, _):
                for u in range(4):
                    o = cb * 64 + u * 16
                    y0 = ybuf[2 * t, pl.ds(o, 16)]
                    y1 = ybuf[2 * t + 1, pl.ds(o, 16)]
                    fv = fbuf[t, pl.ds(o, 16)]
                    r = y0 * a0v + y1 * a1v
                    obuf[t, pl.ds(o, 16)] = jnp.where(routed, r, fv)
                return 0

            return lax.fori_loop(0, D // 64, c_body, 0)

        lax.fori_loop(0, 8, tok_body, 0)
        ocp[p] = pltpu.async_copy(
            obuf, out_hbm.at[pl.ds(tbase + g * 8, 8)], sol[p])
    ocp[0].wait()
    ocp[1].wait()


_sc_combine = functools.partial(
    pl.kernel,
    out_type=jax.ShapeDtypeStruct((T, D), jnp.float32),
    mesh=plsc.VectorSubcoreMesh(core_axis_name="c", subcore_axis_name="s"),
    compiler_params=pltpu.CompilerParams(needs_layout_passes=False),
    scratch_types=[
        pltpu.VMEM((CHUNK,), jnp.int32),
        pltpu.VMEM((CHUNK,), jnp.float32),
        pltpu.VMEM((16, D), jnp.float32),
        pltpu.VMEM((16, D), jnp.float32),
        pltpu.VMEM((8, D), jnp.float32),
        pltpu.VMEM((8, D), jnp.float32),
        pltpu.VMEM((8, D), jnp.float32),
        pltpu.VMEM((8, D), jnp.float32),
        pltpu.SemaphoreType.DMA,
        pltpu.SemaphoreType.DMA,
        pltpu.SemaphoreType.DMA,
        pltpu.SemaphoreType.DMA,
        pltpu.SemaphoreType.DMA,
        pltpu.SemaphoreType.DMA,
    ],
)(_sc_combine_body)


def _tc_fallback_body(x_ref, w1_ref, b1_ref, w2_ref, b2_ref, o_ref):
    h = jnp.dot(x_ref[...], w1_ref[...], preferred_element_type=jnp.float32)
    h = jnp.maximum(h + b1_ref[...], 0.0)
    o_ref[...] = (
        jnp.dot(h, w2_ref[...], preferred_element_type=jnp.float32)
        + b2_ref[...])


def _tc_fallback(x, fW1, fb1, fW2, fb2):
    blk = 256
    return pl.pallas_call(
        _tc_fallback_body,
        grid=(T // blk,),
        in_specs=[
            pl.BlockSpec((blk, D), lambda i: (i, 0)),
            pl.BlockSpec((D, D), lambda i: (0, 0)),
            pl.BlockSpec((1, D), lambda i: (0, 0)),
            pl.BlockSpec((D, D), lambda i: (0, 0)),
            pl.BlockSpec((1, D), lambda i: (0, 0)),
        ],
        out_specs=pl.BlockSpec((blk, D), lambda i: (i, 0)),
        out_shape=jax.ShapeDtypeStruct((T, D), jnp.float32),
    )(x, fW1, fb1.reshape(1, D), fW2, fb2.reshape(1, D))


def _tc_expert_body(xg_ref, w1_ref, b1_ref, w2_ref, b2_ref, y_ref):
    h = jnp.dot(xg_ref[...], w1_ref[0], preferred_element_type=jnp.float32)
    h = jnp.maximum(h + b1_ref[0], 0.0)
    y_ref[...] = (
        jnp.dot(h, w2_ref[0], preferred_element_type=jnp.float32)
        + b2_ref[0])


def _tc_experts(xg, W1, b1, W2, b2):
    return pl.pallas_call(
        _tc_expert_body,
        grid=(E,),
        in_specs=[
            pl.BlockSpec((CAP, D), lambda e: (e, 0)),
            pl.BlockSpec((1, D, D), lambda e: (e, 0, 0)),
            pl.BlockSpec((1, 1, D), lambda e: (e, 0, 0)),
            pl.BlockSpec((1, D, D), lambda e: (e, 0, 0)),
            pl.BlockSpec((1, 1, D), lambda e: (e, 0, 0)),
        ],
        out_specs=pl.BlockSpec((CAP, D), lambda e: (e, 0)),
        out_shape=jax.ShapeDtypeStruct((E * CAP, D), jnp.float32),
    )(xg, W1, b1.reshape(E, 1, D), W2, b2.reshape(E, 1, D))


def kernel(x, W1, b1, W2, b2, fW1, fb1, fW2, fb2, routes):
    routes_flat = routes.reshape(-1)
    xg, comb, accf = _sc_route(routes_flat, x)
    fb = _tc_fallback(x, fW1, fb1, fW2, fb2)
    y = _tc_experts(xg, W1, b1, W2, b2)
    return _sc_combine(y, fb, comb, accf)
